# Initial kernel scaffold; baseline (speedup 1.0000x reference)
import jax
import jax.numpy as jnp
from jax.experimental import pallas as pl


def kernel(W, edge_index, BASIS, eps1, W11, b11, W12, b12, eps2, W21, b21, W22, b22, eps3, W31, b31, W32, b32):
    raise NotImplementedError("write your pallas kernel here")


# speedup vs baseline: 15.4921x; 15.4921x over previous
import functools

import jax
import jax.numpy as jnp
from jax import lax
from jax.experimental import pallas as pl
from jax.experimental.pallas import tpu as pltpu
from jax.experimental.pallas import tpu_sc as plsc

FW = 128
SMAX = 16
NC = 2
NS = 16
G = 128


def _agg_body(ns_static, n, x_hbm, edge_hbm, cfg_hbm, out_hbm,
              src_idx, dst_idx, gb0, gb1, zbuf, cfg_vv, gs0, gs1,
              accum):
    TRP = ns_static["TRP"]
    NCH = ns_static["NCH"]
    ZR = ns_static["ZR"]
    FRB = ns_static["FRB"]
    HALF = n // 2

    core = lax.axis_index("c")
    sub = lax.axis_index("s")
    trash = HALF + sub

    pltpu.sync_copy(cfg_hbm, cfg_vv)
    cfg = cfg_vv[...]
    spc = cfg[0]
    tr0 = cfg[1]

    def _fill(i, _):
        r = i // (G // 16)
        j = i % (G // 16)
        src_idx[r, pl.ds(j * 16, 16)] = jnp.zeros((16,), jnp.int32)
        dst_idx[r, pl.ds(j * 16, 16)] = jnp.full((16,), trash, jnp.int32)
        return 0
    lax.fori_loop(0, NCH * (G // 16), _fill, 0)

    def _stage(c, _):
        R = sub + NS * c

        @pl.when(R < TRP)
        def _():
            off = pl.multiple_of(R * G, G)
            pltpu.sync_copy(edge_hbm.at[0].at[pl.ds(off, G)], src_idx.at[c])
            pltpu.sync_copy(edge_hbm.at[1].at[pl.ds(off, G)], dst_idx.at[c])
        return 0
    lax.fori_loop(0, NCH, _stage, 0)

    def _z(i, _):
        zbuf[i // (FW // 16), pl.ds((i % (FW // 16)) * 16, 16)] = (
            jnp.zeros((16,), jnp.float32))
        return 0
    lax.fori_loop(0, zbuf.shape[0] * (FW // 16), _z, 0)

    def _slice(jj, _):
        s = core * spc + jj
        xs = x_hbm.at[s]
        for p in (0, 1):
            start = 0 if p == 0 else tr0
            end = tr0 if p == 0 else TRP
            pltpu.sync_copy(zbuf, accum.at[pl.ds(sub * ZR, zbuf.shape[0])])
            pltpu.sync_copy(
                zbuf.at[pl.ds(0, ZR - zbuf.shape[0])],
                accum.at[pl.ds(sub * ZR + zbuf.shape[0],
                               ZR - zbuf.shape[0])])
            plsc.subcore_barrier()

            clo = jnp.right_shift(start - sub + (NS - 1), 4)
            chi = jnp.right_shift(end - sub + (NS - 1), 4)
            nch = chi - clo

            @pl.when(nch > 0)
            def _():
                pltpu.async_copy(xs.at[src_idx.at[clo]], gb0, gs0)

            @pl.when(nch > 1)
            def _():
                pltpu.async_copy(xs.at[src_idx.at[clo + 1]], gb1, gs1)

            def _pair(ip, _):
                c0 = clo + 2 * ip
                c1 = c0 + 1
                pltpu.make_async_copy(xs.at[src_idx.at[c0]], gb0, gs0).wait()
                pltpu.sync_copy(gb0, accum.at[dst_idx.at[c0]], add=True)

                @pl.when(c0 + 2 < chi)
                def _():
                    pltpu.async_copy(xs.at[src_idx.at[c0 + 2]], gb0, gs0)

                @pl.when(c1 < chi)
                def _():
                    pltpu.make_async_copy(xs.at[src_idx.at[c1]], gb1,
                                          gs1).wait()
                    pltpu.sync_copy(gb1, accum.at[dst_idx.at[c1]], add=True)

                @pl.when(c1 + 2 < chi)
                def _():
                    pltpu.async_copy(xs.at[src_idx.at[c1 + 2]], gb1, gs1)
                return 0
            lax.fori_loop(0, jnp.right_shift(nch + 1, 1), _pair, 0)
            plsc.subcore_barrier()

            obase = p * HALF
            pltpu.sync_copy(accum.at[pl.ds(sub * FRB, FRB)],
                            out_hbm.at[s].at[pl.ds(obase + sub * FRB, FRB)])
            ext = HALF - NS * FRB
            if ext:
                @pl.when(sub == NS - 1)
                def _():
                    pltpu.sync_copy(
                        accum.at[pl.ds(NS * FRB, ext)],
                        out_hbm.at[s].at[pl.ds(obase + NS * FRB, ext)])
            plsc.subcore_barrier()
        return 0
    lax.fori_loop(0, spc, _slice, 0)


@functools.cache
def _make_agg(n, ep):
    assert ep % G == 0 and n % 2 == 0
    HALF = n // 2
    TRP = ep // G
    NCH = -(-TRP // NS)
    ZR = -(-(HALF + NS) // (NS * 8)) * 8
    ACC = ZR * NS
    ZB = ZR // 2
    assert ZB % 8 == 0
    FRB = (HALF // (NS * 8)) * 8
    cfg = dict(TRP=TRP, NCH=NCH, ZR=ZR, FRB=FRB)
    body = functools.partial(_agg_body, cfg, n)
    return pl.kernel(
        body,
        out_type=jax.ShapeDtypeStruct((SMAX, n, FW), jnp.float32),
        mesh=plsc.VectorSubcoreMesh(core_axis_name="c", subcore_axis_name="s"),
        scratch_types=[
            pltpu.VMEM((NCH, G), jnp.int32),
            pltpu.VMEM((NCH, G), jnp.int32),
            pltpu.VMEM((G, FW), jnp.float32),
            pltpu.VMEM((G, FW), jnp.float32),
            pltpu.VMEM((ZB, FW), jnp.float32),
            pltpu.VMEM((16,), jnp.int32),
            pltpu.SemaphoreType.DMA,
            pltpu.SemaphoreType.DMA,
            pltpu.VMEM_SHARED((ACC, FW), jnp.float32),
        ],
    )


def _edge_prep(edge_index, n):
    src = edge_index[0].astype(jnp.int32)
    dst = edge_index[1].astype(jnp.int32)
    e = src.shape[0]
    half = n // 2
    trp = -(-e // G) + 2
    b = dst >= half
    c1 = jnp.cumsum(b.astype(jnp.int32))
    cnt0 = e - c1[-1]
    tr0 = (cnt0 + G - 1) // G
    i = jnp.arange(e, dtype=jnp.int32)
    pos = jnp.where(b, tr0 * G + c1 - 1, i - c1)
    esrc = jnp.zeros((trp * G,), jnp.int32).at[pos].set(
        src, unique_indices=True)
    edst = jnp.full((trp * G,), half, jnp.int32).at[pos].set(
        dst - b.astype(jnp.int32) * half, unique_indices=True)
    return jnp.stack([esrc, edst]), tr0


def _agg_call(x_s, edges2, tr0, s_act):
    _, n, _ = x_s.shape
    cfgv = (jnp.zeros((16,), jnp.int32)
            .at[0].set(s_act // NC).at[1].set(tr0))
    return _make_agg(n, edges2.shape[1])(x_s, edges2, cfgv)


def _prep_body(x_ref, o_ref):
    s = x_ref.shape[1]
    o_ref[...] = jnp.concatenate(
        [x_ref[:, i, :][None] for i in range(s)], 0)


def _mlp_body(kd, eps_ref, x_ref, a_ref, w1_ref, b1_ref, w2_ref, b2_ref,
              o_ref):
    din, so, dout = kd
    nb = x_ref.shape[1]
    rows_in = FW // din
    rpo = FW // dout
    z = x_ref[0] * eps_ref[0] + a_ref[0]
    zc = jnp.concatenate(
        [z[:, i * din:(i + 1) * din] for i in range(rows_in)], 0)
    h = jnp.maximum(
        jnp.dot(zc, w1_ref[...], preferred_element_type=jnp.float32)
        + b1_ref[...], 0.0)
    o = (jnp.dot(h, w2_ref[...], preferred_element_type=jnp.float32)
         + b2_ref[...])
    parts = []
    for j in range(so):
        seg = [o[(j * rpo + t) * nb:(j * rpo + t + 1) * nb]
               for t in range(rpo)]
        parts.append(jnp.concatenate(seg, 1)[None])
    o_ref[...] = jnp.concatenate(parts, 0)


def _mlp_final_body(eps_ref, x_ref, a_ref, w1_ref, b1_ref, w2_ref, b2_ref,
                    o_ref):
    s, nb, fw = x_ref.shape
    dout = w2_ref.shape[1]
    rpo = fw // dout
    z = x_ref[...] * eps_ref[0] + a_ref[...]
    zc = jnp.concatenate(
        [z[i][:, t * dout:(t + 1) * dout]
         for i in range(s) for t in range(rpo)], 0)
    h = jnp.maximum(
        jnp.dot(zc, w1_ref[...], preferred_element_type=jnp.float32)
        + b1_ref[...], 0.0)
    o = (jnp.dot(h, w2_ref[...], preferred_element_type=jnp.float32)
         + b2_ref[...])
    o_ref[...] = jnp.concatenate(
        [o[r * nb:(r + 1) * nb][:, None, :] for r in range(s * rpo)], 1)


def _mlp_call(x_s, a_s, eps, w1, b1, w2, b2, din, out_sliced, n, k, s_act):
    S = s_act
    dout = w2.shape[1]
    epsv = jnp.reshape(1.0 + eps, (1,))
    rows_in = FW // din
    b1 = b1.reshape(1, -1)
    b2 = b2.reshape(1, -1)
    wspecs = [
        pl.BlockSpec(memory_space=pltpu.SMEM),
        None, None,
        pl.BlockSpec(w1.shape, lambda *a: (0, 0)),
        pl.BlockSpec(b1.shape, lambda *a: (0, 0)),
        pl.BlockSpec(w2.shape, lambda *a: (0, 0)),
        pl.BlockSpec(b2.shape, lambda *a: (0, 0)),
    ]
    if out_sliced:
        NB = 1000
        so = (rows_in * dout) // FW
        xspec = pl.BlockSpec((1, NB, FW), lambda s, b: (s, b, 0))
        wspecs[1] = xspec
        wspecs[2] = xspec
        return pl.pallas_call(
            functools.partial(_mlp_body, (din, so, dout)),
            grid=(S, n // NB),
            in_specs=wspecs,
            out_specs=pl.BlockSpec((so, NB, FW), lambda s, b: (s, b, 0)),
            out_shape=jax.ShapeDtypeStruct((SMAX, n, FW), jnp.float32),
        )(epsv, x_s, a_s, w1, b1, w2, b2)
    NB = 400
    xspec = pl.BlockSpec((SMAX, NB, FW), lambda b: (0, b, 0))
    wspecs[1] = xspec
    wspecs[2] = xspec
    return pl.pallas_call(
        _mlp_final_body,
        grid=(n // NB,),
        in_specs=wspecs,
        out_specs=pl.BlockSpec((NB, k, dout), lambda b: (b, 0, 0)),
        out_shape=jax.ShapeDtypeStruct((n, k, dout), jnp.float32),
    )(epsv, x_s, a_s, w1, b1, w2, b2)


def kernel(W, edge_index, BASIS, eps1, W11, b11, W12, b12,
           eps2, W21, b21, W22, b22, eps3, W31, b31, W32, b32):
    n, k, m = W.shape
    e = edge_index.shape[1]
    S1 = (k * m) // FW
    W4 = W.reshape(n, S1, FW)
    NB = 1000
    x1 = pl.pallas_call(
        _prep_body,
        grid=(n // NB,),
        in_specs=[pl.BlockSpec((NB, S1, FW), lambda b: (b, 0, 0))],
        out_specs=pl.BlockSpec((S1, NB, FW), lambda b: (0, b, 0)),
        out_shape=jax.ShapeDtypeStruct((SMAX, n, FW), jnp.float32),
    )(W4)

    edges2, tr0 = _edge_prep(edge_index, n)
    agg1 = _agg_call(x1, edges2, tr0, S1)
    x2 = _mlp_call(x1, agg1, eps1, W11, b11, W12, b12, m, True, n, k, S1)
    agg2 = _agg_call(x2, edges2, tr0, SMAX)
    x3 = _mlp_call(x2, agg2, eps2, W21, b21, W22, b22, W21.shape[0], True,
                   n, k, SMAX)
    agg3 = _agg_call(x3, edges2, tr0, SMAX)
    return _mlp_call(x3, agg3, eps3, W31, b31, W32, b32, W31.shape[0], False,
                     n, k, SMAX)


# speedup vs baseline: 21.0294x; 1.3574x over previous
import functools

import jax
import jax.numpy as jnp
from jax import lax
from jax.experimental import pallas as pl
from jax.experimental.pallas import tpu as pltpu
from jax.experimental.pallas import tpu_sc as plsc

FW = 128
SMAX = 16
NC = 2
NS = 16
G = 128


def _agg_body(ns_static, n, x_hbm, edge_hbm, cfg_hbm, out_hbm,
              src_idx, dst_idx, gb0, gb1, zbuf, cfg_vv, gs0, gs1,
              accum):
    TRP = ns_static["TRP"]
    NCH = ns_static["NCH"]
    ZR = ns_static["ZR"]
    FRB = ns_static["FRB"]
    HALF = n // 2

    core = lax.axis_index("c")
    sub = lax.axis_index("s")
    trash = HALF + sub

    pltpu.sync_copy(cfg_hbm, cfg_vv)
    cfg = cfg_vv[...]
    spc = cfg[0]
    tr0 = cfg[1]

    def _fill(i, _):
        r = i // (G // 16)
        j = i % (G // 16)
        src_idx[r, pl.ds(j * 16, 16)] = jnp.zeros((16,), jnp.int32)
        dst_idx[r, pl.ds(j * 16, 16)] = jnp.full((16,), trash, jnp.int32)
        return 0
    lax.fori_loop(0, NCH * (G // 16), _fill, 0)

    def _stage(c, _):
        R = sub + NS * c

        @pl.when(R < TRP)
        def _():
            off = pl.multiple_of(R * G, G)
            pltpu.sync_copy(edge_hbm.at[0].at[pl.ds(off, G)], src_idx.at[c])
            pltpu.sync_copy(edge_hbm.at[1].at[pl.ds(off, G)], dst_idx.at[c])
        return 0
    lax.fori_loop(0, NCH, _stage, 0)

    def _z(i, _):
        zbuf[i // (FW // 16), pl.ds((i % (FW // 16)) * 16, 16)] = (
            jnp.zeros((16,), jnp.float32))
        return 0
    lax.fori_loop(0, zbuf.shape[0] * (FW // 16), _z, 0)

    def _slice(jj, _):
        s = core * spc + jj
        xs = x_hbm.at[s]
        for p in (0, 1):
            start = 0 if p == 0 else tr0
            end = tr0 if p == 0 else TRP
            pltpu.sync_copy(zbuf, accum.at[pl.ds(sub * ZR, zbuf.shape[0])])
            pltpu.sync_copy(
                zbuf.at[pl.ds(0, ZR - zbuf.shape[0])],
                accum.at[pl.ds(sub * ZR + zbuf.shape[0],
                               ZR - zbuf.shape[0])])
            plsc.subcore_barrier()

            clo = jnp.right_shift(start - sub + (NS - 1), 4)
            chi = jnp.right_shift(end - sub + (NS - 1), 4)
            nch = chi - clo

            @pl.when(nch > 0)
            def _():
                pltpu.async_copy(xs.at[src_idx.at[clo]], gb0, gs0)

            @pl.when(nch > 1)
            def _():
                pltpu.async_copy(xs.at[src_idx.at[clo + 1]], gb1, gs1)

            def _pair(ip, _):
                c0 = clo + 2 * ip
                c1 = c0 + 1
                pltpu.make_async_copy(xs.at[src_idx.at[c0]], gb0, gs0).wait()
                pltpu.sync_copy(gb0, accum.at[dst_idx.at[c0]], add=True)

                @pl.when(c0 + 2 < chi)
                def _():
                    pltpu.async_copy(xs.at[src_idx.at[c0 + 2]], gb0, gs0)

                @pl.when(c1 < chi)
                def _():
                    pltpu.make_async_copy(xs.at[src_idx.at[c1]], gb1,
                                          gs1).wait()
                    pltpu.sync_copy(gb1, accum.at[dst_idx.at[c1]], add=True)

                @pl.when(c1 + 2 < chi)
                def _():
                    pltpu.async_copy(xs.at[src_idx.at[c1 + 2]], gb1, gs1)
                return 0
            lax.fori_loop(0, jnp.right_shift(nch + 1, 1), _pair, 0)
            plsc.subcore_barrier()

            obase = p * HALF
            pltpu.sync_copy(accum.at[pl.ds(sub * FRB, FRB)],
                            out_hbm.at[s].at[pl.ds(obase + sub * FRB, FRB)])
            ext = HALF - NS * FRB
            if ext:
                @pl.when(sub == NS - 1)
                def _():
                    pltpu.sync_copy(
                        accum.at[pl.ds(NS * FRB, ext)],
                        out_hbm.at[s].at[pl.ds(obase + NS * FRB, ext)])
            plsc.subcore_barrier()
        return 0
    lax.fori_loop(0, spc, _slice, 0)


@functools.cache
def _make_agg(n, ep):
    assert ep % G == 0 and n % 2 == 0
    HALF = n // 2
    TRP = ep // G
    NCH = -(-TRP // NS)
    ZR = -(-(HALF + NS) // (NS * 8)) * 8
    ACC = ZR * NS
    ZB = ZR // 2
    assert ZB % 8 == 0
    FRB = (HALF // (NS * 8)) * 8
    cfg = dict(TRP=TRP, NCH=NCH, ZR=ZR, FRB=FRB)
    body = functools.partial(_agg_body, cfg, n)
    return pl.kernel(
        body,
        out_type=jax.ShapeDtypeStruct((SMAX, n, FW), jnp.float32),
        mesh=plsc.VectorSubcoreMesh(core_axis_name="c", subcore_axis_name="s"),
        scratch_types=[
            pltpu.VMEM((NCH, G), jnp.int32),
            pltpu.VMEM((NCH, G), jnp.int32),
            pltpu.VMEM((G, FW), jnp.float32),
            pltpu.VMEM((G, FW), jnp.float32),
            pltpu.VMEM((ZB, FW), jnp.float32),
            pltpu.VMEM((16,), jnp.int32),
            pltpu.SemaphoreType.DMA,
            pltpu.SemaphoreType.DMA,
            pltpu.VMEM_SHARED((ACC, FW), jnp.float32),
        ],
    )


def _edge_prep(edge_index, n):
    src = edge_index[0].astype(jnp.int32)
    dst = edge_index[1].astype(jnp.int32)
    e = src.shape[0]
    half = n // 2
    trp = -(-e // G) + 2
    b = dst >= half
    c1 = jnp.cumsum(b.astype(jnp.int32))
    cnt0 = e - c1[-1]
    tr0 = (cnt0 + G - 1) // G
    i = jnp.arange(e, dtype=jnp.int32)
    pos = jnp.where(b, tr0 * G + c1 - 1, i - c1)
    inv = jnp.zeros((trp * G,), jnp.int32).at[pos].add(
        i + 1, unique_indices=True)
    jc = jnp.maximum(inv - 1, 0)
    live = inv > 0
    esrc = jnp.where(live, src[jc], 0)
    dstl = dst - b.astype(jnp.int32) * half
    edst = jnp.where(live, dstl[jc], half)
    return jnp.stack([esrc, edst]), tr0


def _agg_call(x_s, edges2, tr0, s_act):
    _, n, _ = x_s.shape
    cfgv = (jnp.zeros((16,), jnp.int32)
            .at[0].set(s_act // NC).at[1].set(tr0))
    return _make_agg(n, edges2.shape[1])(x_s, edges2, cfgv)


def _prep_body(x_ref, o_ref):
    s = x_ref.shape[1]
    o_ref[...] = jnp.concatenate(
        [x_ref[:, i, :][None] for i in range(s)], 0)


def _mlp_body(kd, eps_ref, x_ref, a_ref, w1_ref, b1_ref, w2_ref, b2_ref,
              o_ref):
    din, so, dout = kd
    nb = x_ref.shape[1]
    rows_in = FW // din
    rpo = FW // dout
    z = x_ref[0] * eps_ref[0] + a_ref[0]
    zc = jnp.concatenate(
        [z[:, i * din:(i + 1) * din] for i in range(rows_in)], 0)
    h = jnp.maximum(
        jnp.dot(zc, w1_ref[...], preferred_element_type=jnp.float32)
        + b1_ref[...], 0.0)
    o = (jnp.dot(h, w2_ref[...], preferred_element_type=jnp.float32)
         + b2_ref[...])
    parts = []
    for j in range(so):
        seg = [o[(j * rpo + t) * nb:(j * rpo + t + 1) * nb]
               for t in range(rpo)]
        parts.append(jnp.concatenate(seg, 1)[None])
    o_ref[...] = jnp.concatenate(parts, 0)


def _mlp_final_body(eps_ref, x_ref, a_ref, w1_ref, b1_ref, w2_ref, b2_ref,
                    o_ref):
    s, nb, fw = x_ref.shape
    dout = w2_ref.shape[1]
    rpo = fw // dout
    z = x_ref[...] * eps_ref[0] + a_ref[...]
    zc = jnp.concatenate(
        [z[i][:, t * dout:(t + 1) * dout]
         for i in range(s) for t in range(rpo)], 0)
    h = jnp.maximum(
        jnp.dot(zc, w1_ref[...], preferred_element_type=jnp.float32)
        + b1_ref[...], 0.0)
    o = (jnp.dot(h, w2_ref[...], preferred_element_type=jnp.float32)
         + b2_ref[...])
    o_ref[...] = jnp.concatenate(
        [o[r * nb:(r + 1) * nb][:, None, :] for r in range(s * rpo)], 1)


def _mlp_call(x_s, a_s, eps, w1, b1, w2, b2, din, out_sliced, n, k, s_act):
    S = s_act
    dout = w2.shape[1]
    epsv = jnp.reshape(1.0 + eps, (1,))
    rows_in = FW // din
    b1 = b1.reshape(1, -1)
    b2 = b2.reshape(1, -1)
    wspecs = [
        pl.BlockSpec(memory_space=pltpu.SMEM),
        None, None,
        pl.BlockSpec(w1.shape, lambda *a: (0, 0)),
        pl.BlockSpec(b1.shape, lambda *a: (0, 0)),
        pl.BlockSpec(w2.shape, lambda *a: (0, 0)),
        pl.BlockSpec(b2.shape, lambda *a: (0, 0)),
    ]
    if out_sliced:
        NB = 1000
        so = (rows_in * dout) // FW
        xspec = pl.BlockSpec((1, NB, FW), lambda s, b: (s, b, 0))
        wspecs[1] = xspec
        wspecs[2] = xspec
        return pl.pallas_call(
            functools.partial(_mlp_body, (din, so, dout)),
            grid=(S, n // NB),
            in_specs=wspecs,
            out_specs=pl.BlockSpec((so, NB, FW), lambda s, b: (s, b, 0)),
            out_shape=jax.ShapeDtypeStruct((SMAX, n, FW), jnp.float32),
        )(epsv, x_s, a_s, w1, b1, w2, b2)
    NB = 400
    xspec = pl.BlockSpec((SMAX, NB, FW), lambda b: (0, b, 0))
    wspecs[1] = xspec
    wspecs[2] = xspec
    return pl.pallas_call(
        _mlp_final_body,
        grid=(n // NB,),
        in_specs=wspecs,
        out_specs=pl.BlockSpec((NB, k, dout), lambda b: (b, 0, 0)),
        out_shape=jax.ShapeDtypeStruct((n, k, dout), jnp.float32),
    )(epsv, x_s, a_s, w1, b1, w2, b2)


def kernel(W, edge_index, BASIS, eps1, W11, b11, W12, b12,
           eps2, W21, b21, W22, b22, eps3, W31, b31, W32, b32):
    n, k, m = W.shape
    e = edge_index.shape[1]
    S1 = (k * m) // FW
    W4 = W.reshape(n, S1, FW)
    NB = 1000
    x1 = pl.pallas_call(
        _prep_body,
        grid=(n // NB,),
        in_specs=[pl.BlockSpec((NB, S1, FW), lambda b: (b, 0, 0))],
        out_specs=pl.BlockSpec((S1, NB, FW), lambda b: (0, b, 0)),
        out_shape=jax.ShapeDtypeStruct((SMAX, n, FW), jnp.float32),
    )(W4)

    edges2, tr0 = _edge_prep(edge_index, n)
    agg1 = _agg_call(x1, edges2, tr0, S1)
    x2 = _mlp_call(x1, agg1, eps1, W11, b11, W12, b12, m, True, n, k, S1)
    agg2 = _agg_call(x2, edges2, tr0, SMAX)
    x3 = _mlp_call(x2, agg2, eps2, W21, b21, W22, b22, W21.shape[0], True,
                   n, k, SMAX)
    agg3 = _agg_call(x3, edges2, tr0, SMAX)
    return _mlp_call(x3, agg3, eps3, W31, b31, W32, b32, W31.shape[0], False,
                     n, k, SMAX)


# speedup vs baseline: 22.6777x; 1.0784x over previous
import functools

import jax
import jax.numpy as jnp
from jax import lax
from jax.experimental import pallas as pl
from jax.experimental.pallas import tpu as pltpu
from jax.experimental.pallas import tpu_sc as plsc

FW = 128
SMAX = 16
NC = 2
NS = 16
G = 128


def _agg_body(ns_static, n, x_hbm, edge_hbm, cfg_hbm, out_hbm,
              src_idx, dst_idx, gb0, gb1, gb2, zbuf, cfg_vv,
              gs0, gs1, gs2, accum):
    TRP = ns_static["TRP"]
    NCH = ns_static["NCH"]
    ZR = ns_static["ZR"]
    FRB = ns_static["FRB"]
    HALF = n // 2

    core = lax.axis_index("c")
    sub = lax.axis_index("s")
    trash = HALF + sub

    pltpu.sync_copy(cfg_hbm, cfg_vv)
    cfg = cfg_vv[...]
    spc = cfg[0]
    tr0 = cfg[1]

    def _fill(i, _):
        r = i // (G // 16)
        j = i % (G // 16)
        src_idx[r, pl.ds(j * 16, 16)] = jnp.zeros((16,), jnp.int32)
        dst_idx[r, pl.ds(j * 16, 16)] = jnp.full((16,), trash, jnp.int32)
        return 0
    lax.fori_loop(0, NCH * (G // 16), _fill, 0)

    def _stage(c, _):
        R = sub + NS * c

        @pl.when(R < TRP)
        def _():
            off = pl.multiple_of(R * G, G)
            pltpu.sync_copy(edge_hbm.at[0].at[pl.ds(off, G)], src_idx.at[c])
            pltpu.sync_copy(edge_hbm.at[1].at[pl.ds(off, G)], dst_idx.at[c])
        return 0
    lax.fori_loop(0, NCH, _stage, 0)

    def _z(i, _):
        zbuf[i // (FW // 16), pl.ds((i % (FW // 16)) * 16, 16)] = (
            jnp.zeros((16,), jnp.float32))
        return 0
    lax.fori_loop(0, zbuf.shape[0] * (FW // 16), _z, 0)

    def _slice(jj, _):
        s = core * spc + jj
        xs = x_hbm.at[s]
        for p in (0, 1):
            start = 0 if p == 0 else tr0
            end = tr0 if p == 0 else TRP
            zb = zbuf.shape[0]
            for q in range(ZR // zb):
                pltpu.sync_copy(zbuf,
                                accum.at[pl.ds(sub * ZR + q * zb, zb)])
            plsc.subcore_barrier()

            clo = jnp.right_shift(start - sub + (NS - 1), 4)
            chi = jnp.right_shift(end - sub + (NS - 1), 4)
            nch = chi - clo

            bufs = ((gb0, gs0), (gb1, gs1), (gb2, gs2))
            for j, (gb, gs) in enumerate(bufs):
                @pl.when(nch > j)
                def _(gb=gb, gs=gs, j=j):
                    pltpu.async_copy(xs.at[src_idx.at[clo + j]], gb, gs)

            def _quad(ip, _):
                base = clo + 3 * ip
                for j, (gb, gs) in enumerate(bufs):
                    c = base + j

                    @pl.when(c < chi)
                    def _(c=c, gb=gb, gs=gs):
                        pltpu.make_async_copy(xs.at[src_idx.at[c]], gb,
                                              gs).wait()
                        pltpu.sync_copy(gb, accum.at[dst_idx.at[c]],
                                        add=True)

                    @pl.when(c + 3 < chi)
                    def _(c=c, gb=gb, gs=gs):
                        pltpu.async_copy(xs.at[src_idx.at[c + 3]], gb, gs)
                return 0
            lax.fori_loop(0, lax.div(nch + 2, 3), _quad, 0)
            plsc.subcore_barrier()

            obase = p * HALF
            pltpu.sync_copy(accum.at[pl.ds(sub * FRB, FRB)],
                            out_hbm.at[s].at[pl.ds(obase + sub * FRB, FRB)])
            ext = HALF - NS * FRB
            if ext:
                @pl.when(sub == NS - 1)
                def _():
                    pltpu.sync_copy(
                        accum.at[pl.ds(NS * FRB, ext)],
                        out_hbm.at[s].at[pl.ds(obase + NS * FRB, ext)])
            plsc.subcore_barrier()
        return 0
    lax.fori_loop(0, spc, _slice, 0)


@functools.cache
def _make_agg(n, ep):
    assert ep % G == 0 and n % 2 == 0
    HALF = n // 2
    TRP = ep // G
    NCH = -(-TRP // NS)
    ZR = -(-(HALF + NS) // (NS * 8)) * 8
    ACC = ZR * NS
    ZB = ZR // 4
    assert ZB % 8 == 0 and ZB * 4 == ZR
    FRB = (HALF // (NS * 8)) * 8
    cfg = dict(TRP=TRP, NCH=NCH, ZR=ZR, FRB=FRB)
    body = functools.partial(_agg_body, cfg, n)
    return pl.kernel(
        body,
        out_type=jax.ShapeDtypeStruct((SMAX, n, FW), jnp.float32),
        mesh=plsc.VectorSubcoreMesh(core_axis_name="c", subcore_axis_name="s"),
        scratch_types=[
            pltpu.VMEM((NCH, G), jnp.int32),
            pltpu.VMEM((NCH, G), jnp.int32),
            pltpu.VMEM((G, FW), jnp.float32),
            pltpu.VMEM((G, FW), jnp.float32),
            pltpu.VMEM((G, FW), jnp.float32),
            pltpu.VMEM((ZB, FW), jnp.float32),
            pltpu.VMEM((16,), jnp.int32),
            pltpu.SemaphoreType.DMA,
            pltpu.SemaphoreType.DMA,
            pltpu.SemaphoreType.DMA,
            pltpu.VMEM_SHARED((ACC, FW), jnp.float32),
        ],
    )


def _edge_prep(edge_index, n):
    src = edge_index[0].astype(jnp.int32)
    dst = edge_index[1].astype(jnp.int32)
    e = src.shape[0]
    half = n // 2
    trp = -(-e // G) + 2
    b = dst >= half
    c1 = jnp.cumsum(b.astype(jnp.int32))
    cnt0 = e - c1[-1]
    tr0 = (cnt0 + G - 1) // G
    i = jnp.arange(e, dtype=jnp.int32)
    pos = jnp.where(b, tr0 * G + c1 - 1, i - c1)
    inv = jnp.zeros((trp * G,), jnp.int32).at[pos].add(
        i + 1, unique_indices=True)
    jc = jnp.maximum(inv - 1, 0)
    live = inv > 0
    esrc = jnp.where(live, src[jc], 0)
    dstl = dst - b.astype(jnp.int32) * half
    edst = jnp.where(live, dstl[jc], half)
    return jnp.stack([esrc, edst]), tr0


def _agg_call(x_s, edges2, tr0, s_act):
    _, n, _ = x_s.shape
    cfgv = (jnp.zeros((16,), jnp.int32)
            .at[0].set(s_act // NC).at[1].set(tr0))
    return _make_agg(n, edges2.shape[1])(x_s, edges2, cfgv)


def _prep_body(x_ref, o_ref):
    s = x_ref.shape[1]
    o_ref[...] = jnp.concatenate(
        [x_ref[:, i, :][None] for i in range(s)], 0)


def _mlp_body(kd, eps_ref, x_ref, a_ref, w1_ref, b1_ref, w2_ref, b2_ref,
              o_ref):
    din, so, dout = kd
    nb = x_ref.shape[1]
    rows_in = FW // din
    rpo = FW // dout
    z = x_ref[0] * eps_ref[0] + a_ref[0]
    zc = jnp.concatenate(
        [z[:, i * din:(i + 1) * din] for i in range(rows_in)], 0)
    h = jnp.maximum(
        jnp.dot(zc, w1_ref[...], preferred_element_type=jnp.float32)
        + b1_ref[...], 0.0)
    o = (jnp.dot(h, w2_ref[...], preferred_element_type=jnp.float32)
         + b2_ref[...])
    parts = []
    for j in range(so):
        seg = [o[(j * rpo + t) * nb:(j * rpo + t + 1) * nb]
               for t in range(rpo)]
        parts.append(jnp.concatenate(seg, 1)[None])
    o_ref[...] = jnp.concatenate(parts, 0)


def _mlp_final_body(eps_ref, x_ref, a_ref, w1_ref, b1_ref, w2_ref, b2_ref,
                    o_ref):
    s, nb, fw = x_ref.shape
    dout = w2_ref.shape[1]
    rpo = fw // dout
    z = x_ref[...] * eps_ref[0] + a_ref[...]
    zc = jnp.concatenate(
        [z[i][:, t * dout:(t + 1) * dout]
         for i in range(s) for t in range(rpo)], 0)
    h = jnp.maximum(
        jnp.dot(zc, w1_ref[...], preferred_element_type=jnp.float32)
        + b1_ref[...], 0.0)
    o = (jnp.dot(h, w2_ref[...], preferred_element_type=jnp.float32)
         + b2_ref[...])
    o_ref[...] = jnp.concatenate(
        [o[r * nb:(r + 1) * nb][:, None, :] for r in range(s * rpo)], 1)


def _mlp_call(x_s, a_s, eps, w1, b1, w2, b2, din, out_sliced, n, k, s_act):
    S = s_act
    dout = w2.shape[1]
    epsv = jnp.reshape(1.0 + eps, (1,))
    rows_in = FW // din
    b1 = b1.reshape(1, -1)
    b2 = b2.reshape(1, -1)
    wspecs = [
        pl.BlockSpec(memory_space=pltpu.SMEM),
        None, None,
        pl.BlockSpec(w1.shape, lambda *a: (0, 0)),
        pl.BlockSpec(b1.shape, lambda *a: (0, 0)),
        pl.BlockSpec(w2.shape, lambda *a: (0, 0)),
        pl.BlockSpec(b2.shape, lambda *a: (0, 0)),
    ]
    if out_sliced:
        NB = 1000
        so = (rows_in * dout) // FW
        xspec = pl.BlockSpec((1, NB, FW), lambda s, b: (s, b, 0))
        wspecs[1] = xspec
        wspecs[2] = xspec
        return pl.pallas_call(
            functools.partial(_mlp_body, (din, so, dout)),
            grid=(S, n // NB),
            in_specs=wspecs,
            out_specs=pl.BlockSpec((so, NB, FW), lambda s, b: (s, b, 0)),
            out_shape=jax.ShapeDtypeStruct((SMAX, n, FW), jnp.float32),
        )(epsv, x_s, a_s, w1, b1, w2, b2)
    NB = 400
    xspec = pl.BlockSpec((SMAX, NB, FW), lambda b: (0, b, 0))
    wspecs[1] = xspec
    wspecs[2] = xspec
    return pl.pallas_call(
        _mlp_final_body,
        grid=(n // NB,),
        in_specs=wspecs,
        out_specs=pl.BlockSpec((NB, k, dout), lambda b: (b, 0, 0)),
        out_shape=jax.ShapeDtypeStruct((n, k, dout), jnp.float32),
    )(epsv, x_s, a_s, w1, b1, w2, b2)


def kernel(W, edge_index, BASIS, eps1, W11, b11, W12, b12,
           eps2, W21, b21, W22, b22, eps3, W31, b31, W32, b32):
    n, k, m = W.shape
    e = edge_index.shape[1]
    S1 = (k * m) // FW
    W4 = W.reshape(n, S1, FW)
    NB = 1000
    x1 = pl.pallas_call(
        _prep_body,
        grid=(n // NB,),
        in_specs=[pl.BlockSpec((NB, S1, FW), lambda b: (b, 0, 0))],
        out_specs=pl.BlockSpec((S1, NB, FW), lambda b: (0, b, 0)),
        out_shape=jax.ShapeDtypeStruct((SMAX, n, FW), jnp.float32),
    )(W4)

    edges2, tr0 = _edge_prep(edge_index, n)
    agg1 = _agg_call(x1, edges2, tr0, S1)
    x2 = _mlp_call(x1, agg1, eps1, W11, b11, W12, b12, m, True, n, k, S1)
    agg2 = _agg_call(x2, edges2, tr0, SMAX)
    x3 = _mlp_call(x2, agg2, eps2, W21, b21, W22, b22, W21.shape[0], True,
                   n, k, SMAX)
    agg3 = _agg_call(x3, edges2, tr0, SMAX)
    return _mlp_call(x3, agg3, eps3, W31, b31, W32, b32, W31.shape[0], False,
                     n, k, SMAX)
